# baseline (speedup 1.0000x reference)
import functools

import jax
import jax.numpy as jnp
from jax import lax
from jax.experimental import pallas as pl
from jax.experimental.pallas import tpu as pltpu
from jax.experimental.pallas import tpu_sc as plsc

_CHUNK = 64


def kernel(tokens, W_pos):
    batch, seq_len = tokens.shape
    d_model = W_pos.shape[1]
    info = plsc.get_sparse_core_info()
    nc, ns = info.num_cores, info.num_subcores
    nw = nc * ns
    rows_per_w = (batch * seq_len) // nw
    seq_per_b = seq_len // rows_per_w
    nchunk = rows_per_w // _CHUNK

    mesh = plsc.VectorSubcoreMesh(core_axis_name="c", subcore_axis_name="s")

    @functools.partial(
        pl.kernel,
        mesh=mesh,
        out_type=jax.ShapeDtypeStruct((batch, seq_len, d_model), jnp.float32),
        scratch_types=[
            pltpu.VMEM((_CHUNK, d_model), jnp.float32),
            pltpu.SemaphoreType.DMA,
        ],
    )
    def _sc_copy(w_hbm, out_hbm, buf, sem):
        wid = lax.axis_index("s") * nc + lax.axis_index("c")
        b = wid // seq_per_b
        s0 = (wid % seq_per_b) * rows_per_w
        for j in range(nchunk):
            pltpu.sync_copy(w_hbm.at[pl.ds(s0 + j * _CHUNK, _CHUNK)], buf)
            pltpu.sync_copy(buf, out_hbm.at[b, pl.ds(s0 + j * _CHUNK, _CHUNK)])

    return _sc_copy(W_pos)
